# baseline (speedup 1.0000x reference)
import functools

import jax
import jax.numpy as jnp
from jax import lax
from jax.experimental import pallas as pl
from jax.experimental.pallas import tpu as pltpu
from jax.experimental.pallas import tpu_sc as plsc

_B = 16384 * 50
_D = 32
_NC = 2
_NS = 16
_NW = _NC * _NS
_B_PER_W = _B // _NW
_CHUNK = 3200
_N_CHUNKS = _B_PER_W // _CHUNK
_NSTREAM = 8
_SUB = _CHUNK // _NSTREAM


@functools.partial(
    pl.kernel,
    out_type=jax.ShapeDtypeStruct((_B, _D), jnp.float32),
    mesh=plsc.VectorSubcoreMesh(core_axis_name="c", subcore_axis_name="s"),
    scratch_types=[
        pltpu.VMEM((_CHUNK,), jnp.int32),
        pltpu.VMEM((_CHUNK, _D), jnp.float32),
        pltpu.SemaphoreType.DMA,
    ],
    compiler_params=pltpu.CompilerParams(use_tc_tiling_on_sc=False),
)
def _gather_kernel(idx_hbm, table_hbm, out_hbm, idx_v, rows_v, gsem):
    wid = lax.axis_index("s") * _NC + lax.axis_index("c")
    base = wid * _B_PER_W

    @pl.loop(0, _N_CHUNKS)
    def _chunk(i):
        off = base + i * _CHUNK
        pltpu.sync_copy(idx_hbm.at[pl.ds(off, _CHUNK)], idx_v)
        for g in range(_NSTREAM):
            pltpu.async_copy(
                table_hbm.at[idx_v.at[pl.ds(g * _SUB, _SUB)]],
                rows_v.at[pl.ds(g * _SUB, _SUB)], gsem)
        pltpu.make_async_copy(table_hbm.at[idx_v], rows_v, gsem).wait()
        pltpu.sync_copy(rows_v, out_hbm.at[pl.ds(off, _CHUNK)])


def kernel(input, table):
    flat_idx = input.reshape(-1).astype(jnp.int32)
    out = _gather_kernel(flat_idx, table)
    return out.reshape(input.shape + (table.shape[-1],))


# speedup vs baseline: 1.6202x; 1.6202x over previous
import functools

import jax
import jax.numpy as jnp
from jax import lax
from jax.experimental import pallas as pl
from jax.experimental.pallas import tpu as pltpu
from jax.experimental.pallas import tpu_sc as plsc

_BATCH = 16384
_HIST = 50
_D = 32
_NC = 2
_NS = 16
_NW = _NC * _NS
_ROWS_PER_W = _BATCH // _NW
_CHUNK_ROWS = 64
_CHUNK = _CHUNK_ROWS * _HIST
_N_CHUNKS = _ROWS_PER_W // _CHUNK_ROWS
_NSTREAM = 8
_SUB = _CHUNK // _NSTREAM


@functools.partial(
    pl.kernel,
    out_type=jax.ShapeDtypeStruct((_BATCH, _HIST, _D), jnp.float32),
    mesh=plsc.VectorSubcoreMesh(core_axis_name="c", subcore_axis_name="s"),
    scratch_types=[
        pltpu.VMEM((_CHUNK_ROWS, _HIST), jnp.int32),
        pltpu.VMEM((_CHUNK_ROWS, _HIST, _D), jnp.float32),
        pltpu.SemaphoreType.DMA,
    ],
    compiler_params=pltpu.CompilerParams(use_tc_tiling_on_sc=False),
)
def _gather_kernel(idx_hbm, table_hbm, out_hbm, idx_v, rows_v, gsem):
    wid = lax.axis_index("s") * _NC + lax.axis_index("c")
    base = wid * _ROWS_PER_W

    @pl.loop(0, _N_CHUNKS)
    def _chunk(i):
        row0 = base + i * _CHUNK_ROWS
        pltpu.sync_copy(idx_hbm.at[pl.ds(row0, _CHUNK_ROWS), :], idx_v)

        @pl.loop(0, _CHUNK_ROWS, unroll=8)
        def _row(r):
            pltpu.async_copy(table_hbm.at[idx_v.at[r, :]], rows_v.at[r],
                             gsem)

        pltpu.make_async_copy(out_hbm.at[pl.ds(row0, _CHUNK_ROWS)], rows_v,
                              gsem).wait()
        pltpu.sync_copy(rows_v, out_hbm.at[pl.ds(row0, _CHUNK_ROWS), :, :])


def kernel(input, table):
    return _gather_kernel(input.astype(jnp.int32), table)
